# Initial kernel scaffold; baseline (speedup 1.0000x reference)
import jax
import jax.numpy as jnp
from jax.experimental import pallas as pl


def kernel(samples, component_weight, feat_weight, value_weight):
    raise NotImplementedError("write your pallas kernel here")


# speedup vs baseline: 5.0382x; 5.0382x over previous
import jax
import jax.numpy as jnp
from jax.experimental import pallas as pl
from jax.experimental.pallas import tpu as pltpu

_B, _S = 8, 32
_NCH, _NFEAT, _D = 4, 32, 2048
_NLEV = 100
_R = _B * _S * _NCH


def _flow_body(samples_ref, cw_ref, fw_ref, vw_ref, s4_ref):
    samples = samples_ref[...]
    idxf = jnp.round((samples + 1.0) / 2.0 * 99.0)
    idx = jnp.clip(idxf, 0.0, 99.0).astype(jnp.int32)

    fw = fw_ref[...]
    vw = vw_ref[...]
    iota_l = jax.lax.broadcasted_iota(jnp.int32, (1, _NLEV), 1)

    ht = jnp.zeros((_R, _D), jnp.float32)
    for f in range(_NFEAT):
        col = idx[:, f:f + 1]
        oh = (col == iota_l).astype(jnp.bfloat16)
        vwf = (vw * fw[f:f + 1, :]).astype(jnp.bfloat16)
        ht = ht + jax.lax.dot(oh, vwf, preferred_element_type=jnp.float32)

    cw = cw_ref[...]
    csum = (cw[0:1] + cw[1:2] + cw[2:3] + cw[3:4])
    s4_ref[...] = (ht * csum).reshape(_B, _S, _NCH, _D)


def kernel(samples, component_weight, feat_weight, value_weight):
    samples_r = samples.reshape(_R, _NFEAT)
    s4 = pl.pallas_call(
        _flow_body,
        out_shape=jax.ShapeDtypeStruct((_B, _S, _NCH, _D), jnp.float32),
    )(samples_r, component_weight, feat_weight, value_weight)

    t_interp = jax.nn.sigmoid(s4[:, :, 2, :] + s4[:, :, 3, :])
    h = s4[:, :, 0, :] * (1.0 - t_interp) + t_interp * s4[:, :, 1, :]
    h = jnp.roll(h, shift=1, axis=0)
    h = h.at[0].set(jnp.zeros_like(h[0]))
    s4 = s4 + h[:, :, None, :]
    return jnp.sign(jnp.sum(s4.reshape(_B, _S, -1), axis=1))
